# baseline (speedup 1.0000x reference)
import functools

import jax
import jax.numpy as jnp
from jax import lax
from jax.experimental import pallas as pl
from jax.experimental.pallas import tpu as pltpu
from jax.experimental.pallas import tpu_sc as plsc

_G = 2


def _gather_sc(table2, idx, B, D):
    info = plsc.get_sparse_core_info()
    NC = info.num_cores
    NW = NC * info.num_subcores
    b_per_w = B // NW
    CH = 128
    n_ch = b_per_w // CH

    mesh = plsc.VectorSubcoreMesh(core_axis_name="c", subcore_axis_name="s")

    @functools.partial(
        pl.kernel,
        mesh=mesh,
        out_type=jax.ShapeDtypeStruct((B, D), jnp.float32),
        scratch_types=[
            pltpu.VMEM((b_per_w,), jnp.int32),
            pltpu.VMEM((b_per_w,), jnp.int32),
            pltpu.VMEM((CH, _G * D), jnp.float32),
            pltpu.VMEM((b_per_w, D), jnp.float32),
            pltpu.SemaphoreType.DMA,
            pltpu.SemaphoreType.DMA,
        ],
        compiler_params=pltpu.CompilerParams(needs_layout_passes=False),
    )
    def k(table_hbm, idx_hbm, out_hbm, idx_v, blk_idx, blkbuf, rows_v, sem_i, sem_g):
        wid = lax.axis_index("s") * NC + lax.axis_index("c")
        base = wid * b_per_w
        pltpu.async_copy(idx_hbm.at[pl.ds(base, b_per_w)], idx_v, sem_i).wait()

        def to_blk(g, carry):
            v = idx_v[pl.ds(g * 16, 16)]
            blk_idx[pl.ds(g * 16, 16)] = lax.shift_right_logical(v, 1)
            return carry

        lax.fori_loop(0, b_per_w // 16, to_blk, 0)

        for c in range(n_ch):
            pltpu.async_copy(
                table_hbm.at[blk_idx.at[pl.ds(c * CH, CH)]], blkbuf, sem_g
            ).wait()

            def select(g, carry, c=c):
                v = idx_v[pl.ds(c * CH + g * 16, 16)]
                half_off = lax.bitwise_and(v, jnp.int32(_G - 1)) * D
                slot = lax.iota(jnp.int32, 16) + g * 16
                out_row = slot + c * CH
                for col in range(D):
                    col_v = jnp.full((16,), col, jnp.int32)
                    vals = plsc.load_gather(blkbuf, [slot, half_off + col_v])
                    plsc.store_scatter(rows_v, [out_row, col_v], vals)
                return carry

            lax.fori_loop(0, CH // 16, select, 0)

        pltpu.sync_copy(rows_v, out_hbm.at[pl.ds(base, b_per_w)])

    return k(table2, idx)


def kernel(input, indices):
    B = indices.shape[0]
    V, D = input.shape
    table2 = input.reshape(V // _G, _G * D)
    return _gather_sc(table2, indices.astype(jnp.int32), B, D)


# speedup vs baseline: 1.8306x; 1.8306x over previous
import functools

import jax
import jax.numpy as jnp
from jax import lax
from jax.experimental import pallas as pl
from jax.experimental.pallas import tpu as pltpu
from jax.experimental.pallas import tpu_sc as plsc


def _gather_sc(table, idx, B, D):
    info = plsc.get_sparse_core_info()
    NC = info.num_cores
    NW = NC * info.num_subcores
    b_per_w = B // NW

    mesh = plsc.VectorSubcoreMesh(core_axis_name="c", subcore_axis_name="s")

    @functools.partial(
        pl.kernel,
        mesh=mesh,
        out_type=jax.ShapeDtypeStruct((B, D), jnp.float32),
        scratch_types=[
            pltpu.VMEM((b_per_w,), jnp.int32),
            pltpu.VMEM((b_per_w, D), jnp.float32),
            pltpu.SemaphoreType.DMA,
            pltpu.SemaphoreType.DMA,
        ],
        compiler_params=pltpu.CompilerParams(needs_layout_passes=False),
    )
    def k(table_hbm, idx_hbm, out_hbm, idx_v, rows_v, sem_i, sem_g):
        wid = lax.axis_index("s") * NC + lax.axis_index("c")
        base = wid * b_per_w
        pltpu.async_copy(idx_hbm.at[pl.ds(base, b_per_w)], idx_v, sem_i).wait()

        def grp(g, carry):
            v = idx_v[pl.ds(g * 16, 16)]
            for j in range(16):
                pltpu.async_copy(
                    table_hbm.at[v[j]], rows_v.at[g * 16 + j], sem_g
                )
            return carry

        lax.fori_loop(0, b_per_w // 16, grp, 0)

        pltpu.make_async_copy(
            table_hbm.at[pl.ds(0, b_per_w)], rows_v, sem_g
        ).wait()

        pltpu.sync_copy(rows_v, out_hbm.at[pl.ds(base, b_per_w)])

    return k(table, idx)


def kernel(input, indices):
    B = indices.shape[0]
    V, D = input.shape
    return _gather_sc(input, indices.astype(jnp.int32), B, D)


# speedup vs baseline: 2.6955x; 1.4725x over previous
import functools

import jax
import jax.numpy as jnp
from jax import lax
from jax.experimental import pallas as pl
from jax.experimental.pallas import tpu as pltpu
from jax.experimental.pallas import tpu_sc as plsc


def _gather_sc(table, idx, B, D):
    info = plsc.get_sparse_core_info()
    NC = info.num_cores
    NW = NC * info.num_subcores
    b_per_w = B // NW

    mesh = plsc.VectorSubcoreMesh(core_axis_name="c", subcore_axis_name="s")

    @functools.partial(
        pl.kernel,
        mesh=mesh,
        out_type=jax.ShapeDtypeStruct((B, D), jnp.float32),
        scratch_types=[
            pltpu.VMEM((b_per_w,), jnp.int32),
            pltpu.VMEM((b_per_w, D), jnp.float32),
            pltpu.SemaphoreType.DMA,
            pltpu.SemaphoreType.DMA,
        ],
        compiler_params=pltpu.CompilerParams(needs_layout_passes=False),
    )
    def k(table_hbm, idx_hbm, out_hbm, idx_v, rows_v, sem_i, sem_g):
        wid = lax.axis_index("s") * NC + lax.axis_index("c")
        base = wid * b_per_w
        pltpu.async_copy(idx_hbm.at[pl.ds(base, b_per_w)], idx_v, sem_i).wait()

        def grp(g, carry):
            v = idx_v[pl.ds(g * 16, 16)]
            blk = lax.shift_right_logical(v, 3)
            sub = lax.bitwise_and(v, jnp.int32(7))
            for j in range(16):
                pltpu.async_copy(
                    table_hbm.at[blk[j], sub[j]], rows_v.at[g * 16 + j], sem_g
                )
            return carry

        lax.fori_loop(0, b_per_w // 16, grp, 0)

        def drain(g, carry):
            pltpu.make_async_copy(
                table_hbm.at[0, 0], rows_v.at[0], sem_g
            ).wait()
            return carry

        lax.fori_loop(0, b_per_w, drain, 0)

        pltpu.sync_copy(rows_v, out_hbm.at[pl.ds(base, b_per_w)])

    return k(table, idx)


def kernel(input, indices):
    B = indices.shape[0]
    V, D = input.shape
    table3 = input.reshape(V // 8, 8, D)
    return _gather_sc(table3, indices.astype(jnp.int32), B, D)


# speedup vs baseline: 2.6984x; 1.0011x over previous
import functools

import jax
import jax.numpy as jnp
from jax import lax
from jax.experimental import pallas as pl
from jax.experimental.pallas import tpu as pltpu
from jax.experimental.pallas import tpu_sc as plsc


def _gather_sc(table, idx, B, D):
    info = plsc.get_sparse_core_info()
    NC = info.num_cores
    NW = NC * info.num_subcores
    b_per_w = B // NW

    mesh = plsc.VectorSubcoreMesh(core_axis_name="c", subcore_axis_name="s")

    @functools.partial(
        pl.kernel,
        mesh=mesh,
        out_type=jax.ShapeDtypeStruct((B, D), jnp.float32),
        scratch_types=[
            pltpu.VMEM((b_per_w,), jnp.int32),
            pltpu.VMEM((b_per_w, D), jnp.float32),
            pltpu.SemaphoreType.DMA,
            pltpu.SemaphoreType.DMA,
        ],
        compiler_params=pltpu.CompilerParams(
            needs_layout_passes=False, use_tc_tiling_on_sc=True
        ),
    )
    def k(table_hbm, idx_hbm, out_hbm, idx_v, rows_v, sem_i, sem_g):
        wid = lax.axis_index("s") * NC + lax.axis_index("c")
        base = wid * b_per_w
        pltpu.async_copy(idx_hbm.at[pl.ds(base, b_per_w)], idx_v, sem_i).wait()

        def grp(g, carry):
            v = idx_v[pl.ds(g * 16, 16)]
            blk = lax.shift_right_logical(v, 3)
            sub = lax.bitwise_and(v, jnp.int32(7))
            for j in range(16):
                pltpu.async_copy(
                    table_hbm.at[blk[j], sub[j]], rows_v.at[g * 16 + j], sem_g
                )
            return carry

        lax.fori_loop(0, b_per_w // 16, grp, 0)

        def drain(g, carry):
            pltpu.make_async_copy(
                table_hbm.at[0, 0], rows_v.at[0], sem_g
            ).wait()
            return carry

        lax.fori_loop(0, b_per_w, drain, 0)

        pltpu.sync_copy(rows_v, out_hbm.at[pl.ds(base, b_per_w)])

    return k(table, idx)


def kernel(input, indices):
    B = indices.shape[0]
    V, D = input.shape
    table3 = input.reshape(V // 8, 8, D)
    return _gather_sc(table3, indices.astype(jnp.int32), B, D)


# speedup vs baseline: 2.7119x; 1.0050x over previous
import functools

import jax
import jax.numpy as jnp
from jax import lax
from jax.experimental import pallas as pl
from jax.experimental.pallas import tpu as pltpu
from jax.experimental.pallas import tpu_sc as plsc


def _gather_sc(table3, idx, B, D):
    info = plsc.get_sparse_core_info()
    NC = info.num_cores
    NW = NC * info.num_subcores
    b_per_w = B // NW

    mesh = plsc.VectorSubcoreMesh(core_axis_name="c", subcore_axis_name="s")

    @functools.partial(
        pl.kernel,
        mesh=mesh,
        out_type=jax.ShapeDtypeStruct((B, D), jnp.float32),
        scratch_types=[
            pltpu.VMEM((b_per_w,), jnp.int32),
            pltpu.VMEM((b_per_w, D), jnp.float32),
            pltpu.SemaphoreType.DMA,
            pltpu.SemaphoreType.DMA,
        ],
        compiler_params=pltpu.CompilerParams(needs_layout_passes=False),
    )
    def k(table_hbm, idx_hbm, out_hbm, idx_v, rows_v, sem_i, sem_g):
        wid = lax.axis_index("s") * NC + lax.axis_index("c")
        base = wid * b_per_w
        pltpu.async_copy(idx_hbm.at[pl.ds(base, b_per_w)], idx_v, sem_i).wait()

        def grp(g, carry):
            v = idx_v[pl.ds(g * 16, 16)]
            blk = lax.shift_right_logical(v, 3)
            sub = lax.bitwise_and(v, jnp.int32(7))
            for j in range(16):
                pltpu.async_copy(
                    table_hbm.at[blk[j], sub[j]], rows_v.at[g * 16 + j], sem_g
                )
            return carry

        lax.fori_loop(0, b_per_w // 16, grp, 0)

        pltpu.make_async_copy(
            out_hbm.at[pl.ds(0, b_per_w)], rows_v, sem_g
        ).wait()

        pltpu.sync_copy(rows_v, out_hbm.at[pl.ds(base, b_per_w)])

    return k(table3, idx)


def kernel(input, indices):
    B = indices.shape[0]
    V, D = input.shape
    table3 = input.reshape(V // 8, 8, D)
    return _gather_sc(table3, indices.astype(jnp.int32), B, D)
